# baseline (speedup 1.0000x reference)
import functools

import jax
import jax.numpy as jnp
from jax import lax
from jax.experimental import pallas as pl
from jax.experimental.pallas import tpu as pltpu
from jax.experimental.pallas import tpu_sc as plsc

N = 10000
E = 320000
D = 128
NPAD = 10240
NC, NS = 2, 16
NW = NC * NS
CHUNK = 128
CPW = 80
EPW = CPW * CHUNK
EPAD = NW * EPW
ROWS_T = NPAD // NS


def _mesh():
    return plsc.VectorSubcoreMesh(
        core_axis_name="c", subcore_axis_name="s", num_cores=NC, num_subcores=NS
    )


DH = D // NC
CPT = EPAD // CHUNK // NS


@functools.partial(
    pl.kernel,
    out_type=jax.ShapeDtypeStruct((NC, NPAD, DH), jnp.float32),
    mesh=_mesh(),
    compiler_params=pltpu.CompilerParams(use_tc_tiling_on_sc=False),
    scratch_types=[
        pltpu.VMEM((CPT // 2, CHUNK), jnp.int32),
        pltpu.VMEM((4, CHUNK), jnp.int32),
        pltpu.VMEM((4, CHUNK), jnp.int32),
        pltpu.VMEM((CHUNK, DH), jnp.float32),
        pltpu.VMEM((CHUNK, DH), jnp.float32),
        pltpu.VMEM((CHUNK, DH), jnp.float32),
        pltpu.VMEM((CHUNK, DH), jnp.float32),
        pltpu.VMEM_SHARED((NPAD, DH), jnp.float32),
        pltpu.VMEM_SHARED((NPAD, DH), jnp.float32),
        pltpu.SemaphoreType.DMA,
        pltpu.SemaphoreType.DMA,
        pltpu.SemaphoreType.DMA,
        pltpu.SemaphoreType.DMA,
        pltpu.SemaphoreType.DMA,
        pltpu.SemaphoreType.DMA,
        pltpu.SemaphoreType.DMA,
        pltpu.SemaphoreType.DMA,
    ],
)
def _agg_col(g_hbm, pk_hbm, zero_hbm, out_hbm,
             pk_v, src_v, dst_v, rows0, rows1, rows2, rows3,
             g_sh, acc_sh, gs0, gs1, gs2, gs3, ss0, ss1, ss2, ss3):
    c = lax.axis_index("c")
    s = lax.axis_index("s")
    row0 = s * ROWS_T
    HALF = CPT // 2
    pltpu.sync_copy(zero_hbm.at[pl.ds(row0, ROWS_T)],
                    acc_sh.at[pl.ds(row0, ROWS_T)])
    pltpu.sync_copy(g_hbm.at[c, pl.ds(row0, ROWS_T)],
                    g_sh.at[pl.ds(row0, ROWS_T)])
    pltpu.sync_copy(pk_hbm.at[s, pl.ds(0, HALF)], pk_v)

    rows = (rows0, rows1, rows2, rows3)
    gsem = (gs0, gs1, gs2, gs3)
    ssem = (ss0, ss1, ss2, ss3)

    def unpack(ci, k):
        ri = jnp.where(ci >= HALF, ci - HALF, ci)
        for j in range(CHUNK // 16):
            p = pk_v[ri, pl.ds(16 * j, 16)]
            src_v[k, pl.ds(16 * j, 16)] = p & (2**14 - 1)
            dst_v[k, pl.ds(16 * j, 16)] = p >> 14

    plsc.subcore_barrier()

    for k in range(4):
        unpack(jnp.int32(k), k)
        pltpu.async_copy(g_sh.at[src_v.at[k]], rows[k], gsem[k])

    def body(i, carry):
        base = 4 * i

        @pl.when(base + 4 == HALF)
        def _reload():
            pltpu.sync_copy(pk_hbm.at[s, pl.ds(HALF, HALF)], pk_v)

        for k in range(4):
            pltpu.make_async_copy(g_sh.at[src_v.at[k]], rows[k], gsem[k]).wait()
            pltpu.async_copy(rows[k], acc_sh.at[dst_v.at[k]], ssem[k], add=True)
        for k in range(4):
            nxt = jnp.minimum(base + 4 + k, CPT - 1)
            pltpu.make_async_copy(rows[k], acc_sh.at[dst_v.at[k]], ssem[k]).wait()
            unpack(nxt, k)
            pltpu.async_copy(g_sh.at[src_v.at[k]], rows[k], gsem[k])
        return carry

    lax.fori_loop(0, CPT // 4, body, 0)
    for k in range(4):
        pltpu.make_async_copy(g_sh.at[src_v.at[k]], rows[k], gsem[k]).wait()
    plsc.subcore_barrier()
    pltpu.sync_copy(acc_sh.at[pl.ds(row0, ROWS_T)],
                    out_hbm.at[c, pl.ds(row0, ROWS_T)])


@functools.partial(
    pl.kernel,
    out_type=jax.ShapeDtypeStruct((NC, NPAD, 16), jnp.float32),
    mesh=_mesh(),
    compiler_params=pltpu.CompilerParams(use_tc_tiling_on_sc=False),
    scratch_types=[
        pltpu.VMEM((CPW, CHUNK), jnp.int32),
        pltpu.VMEM((CHUNK,), jnp.int32),
        pltpu.VMEM((CHUNK,), jnp.int32),
        pltpu.VMEM((CHUNK,), jnp.int32),
        pltpu.VMEM((CHUNK,), jnp.int32),
        pltpu.VMEM((CHUNK, 16), jnp.float32),
        pltpu.VMEM((CHUNK, 16), jnp.float32),
        pltpu.VMEM_SHARED((NPAD, 16), jnp.float32),
        pltpu.VMEM_SHARED((NPAD, 16), jnp.float32),
        pltpu.SemaphoreType.DMA,
        pltpu.SemaphoreType.DMA,
        pltpu.SemaphoreType.DMA,
        pltpu.SemaphoreType.DMA,
    ],
)
def _AGG16(g_hbm, pk_hbm, zero_hbm, out_hbm,
           pk_v, src_a, dst_a, src_b, dst_b, rows_a, rows_b,
           g_sh, acc_sh, gsa, gsb, ssa, ssb):
    c = lax.axis_index("c")
    s = lax.axis_index("s")
    w = s * NC + c
    row0 = s * ROWS_T
    pltpu.sync_copy(zero_hbm.at[pl.ds(row0, ROWS_T)],
                    acc_sh.at[pl.ds(row0, ROWS_T)])
    pltpu.sync_copy(g_hbm.at[pl.ds(row0, ROWS_T)],
                    g_sh.at[pl.ds(row0, ROWS_T)])
    pltpu.sync_copy(pk_hbm.at[w], pk_v)

    def unpack(ci, src_buf, dst_buf):
        for j in range(CHUNK // 16):
            p = pk_v[ci, pl.ds(16 * j, 16)]
            src_buf[pl.ds(16 * j, 16)] = p & (2**14 - 1)
            dst_buf[pl.ds(16 * j, 16)] = p >> 14

    plsc.subcore_barrier()

    unpack(0, src_a, dst_a)
    unpack(1, src_b, dst_b)
    pltpu.async_copy(g_sh.at[src_a], rows_a, gsa)
    pltpu.async_copy(g_sh.at[src_b], rows_b, gsb)

    def body(i, carry):
        a2 = jnp.minimum(2 * i + 2, CPW - 1)
        b2 = jnp.minimum(2 * i + 3, CPW - 1)
        pltpu.make_async_copy(g_sh.at[src_a], rows_a, gsa).wait()
        pltpu.async_copy(rows_a, acc_sh.at[dst_a], ssa, add=True)
        pltpu.make_async_copy(g_sh.at[src_b], rows_b, gsb).wait()
        pltpu.make_async_copy(rows_a, acc_sh.at[dst_a], ssa).wait()
        unpack(a2, src_a, dst_a)
        pltpu.async_copy(g_sh.at[src_a], rows_a, gsa)
        pltpu.async_copy(rows_b, acc_sh.at[dst_b], ssb, add=True)
        pltpu.make_async_copy(rows_b, acc_sh.at[dst_b], ssb).wait()
        unpack(b2, src_b, dst_b)
        pltpu.async_copy(g_sh.at[src_b], rows_b, gsb)
        return carry

    lax.fori_loop(0, CPW // 2, body, 0)
    pltpu.make_async_copy(g_sh.at[src_a], rows_a, gsa).wait()
    pltpu.make_async_copy(g_sh.at[src_b], rows_b, gsb).wait()
    plsc.subcore_barrier()
    pltpu.sync_copy(acc_sh.at[pl.ds(row0, ROWS_T)],
                    out_hbm.at[c, pl.ds(row0, ROWS_T)])


@functools.partial(
    pl.kernel,
    out_type=jax.ShapeDtypeStruct((NC, NPAD, 16), jnp.float32),
    mesh=_mesh(),
    compiler_params=pltpu.CompilerParams(use_tc_tiling_on_sc=False),
    scratch_types=[
        pltpu.VMEM((CPW, CHUNK), jnp.int32),
        pltpu.VMEM((CHUNK, 16), jnp.float32),
        pltpu.VMEM_SHARED((NPAD, 16), jnp.float32),
    ],
)
def _deg(dst_hbm, ones_hbm, zero_hbm, out_hbm, dst_v, ones_v, acc_sh):
    c = lax.axis_index("c")
    s = lax.axis_index("s")
    w = s * NC + c
    pltpu.sync_copy(zero_hbm.at[pl.ds(s * ROWS_T, ROWS_T)],
                    acc_sh.at[pl.ds(s * ROWS_T, ROWS_T)])
    pltpu.sync_copy(ones_hbm, ones_v)
    pltpu.sync_copy(dst_hbm.at[w], dst_v)
    plsc.subcore_barrier()

    def body(i, carry):
        pltpu.sync_copy(ones_v, acc_sh.at[dst_v.at[i]], add=True)
        return carry

    lax.fori_loop(0, CPW, body, 0)
    plsc.subcore_barrier()
    pltpu.sync_copy(acc_sh.at[pl.ds(s * ROWS_T, ROWS_T)],
                    out_hbm.at[c, pl.ds(s * ROWS_T, ROWS_T)])


BLK = 512
GRID = NPAD // BLK


def _tc_first(x_pad, d0, d1, W0):
    def body(x_ref, d0_ref, d1_ref, w_ref, g_ref, dis_ref):
        dis = lax.rsqrt(1.0 + d0_ref[...] + d1_ref[...])
        h = lax.dot_general(x_ref[...], w_ref[...], (((1,), (1,)), ((), ())),
                            preferred_element_type=jnp.float32)
        hd = h * dis
        g_ref[...] = jnp.stack([hd[:, :DH], hd[:, DH:]])
        dis_ref[...] = dis

    return pl.pallas_call(
        body,
        grid=(GRID,),
        in_specs=[
            pl.BlockSpec((BLK, D), lambda i: (i, 0)),
            pl.BlockSpec((BLK, 1), lambda i: (i, 0)),
            pl.BlockSpec((BLK, 1), lambda i: (i, 0)),
            pl.BlockSpec((D, D), lambda i: (0, 0)),
        ],
        out_specs=[
            pl.BlockSpec((NC, BLK, DH), lambda i: (0, i, 0)),
            pl.BlockSpec((BLK, 1), lambda i: (i, 0)),
        ],
        out_shape=[
            jax.ShapeDtypeStruct((NC, NPAD, DH), jnp.float32),
            jax.ShapeDtypeStruct((NPAD, 1), jnp.float32),
        ],
    )(x_pad, d0, d1, W0)


def _make_tc_mid(dout):
    def body(a0_ref, a1_ref, g0_ref, g1_ref, dis_ref, b_ref, w_ref, o_ref):
        dis = dis_ref[...]
        t = jnp.concatenate([a0_ref[...] + g0_ref[...],
                             a1_ref[...] + g1_ref[...]], axis=1)
        x = jnp.maximum(dis * t + b_ref[...], 0.0)
        h = lax.dot_general(x, w_ref[...], (((1,), (1,)), ((), ())),
                            preferred_element_type=jnp.float32)
        hd = h * dis
        if dout == D:
            o_ref[...] = jnp.stack([hd[:, :DH], hd[:, DH:]])
        else:
            o_ref[...] = hd

    def run(a0, a1, g0, g1, dis, b, W):
        return pl.pallas_call(
            body,
            grid=(GRID,),
            in_specs=[
                pl.BlockSpec((BLK, DH), lambda i: (i, 0)),
                pl.BlockSpec((BLK, DH), lambda i: (i, 0)),
                pl.BlockSpec((BLK, DH), lambda i: (i, 0)),
                pl.BlockSpec((BLK, DH), lambda i: (i, 0)),
                pl.BlockSpec((BLK, 1), lambda i: (i, 0)),
                pl.BlockSpec((1, D), lambda i: (0, 0)),
                pl.BlockSpec((dout, D), lambda i: (0, 0)),
            ],
            out_specs=(pl.BlockSpec((NC, BLK, DH), lambda i: (0, i, 0))
                       if dout == D else
                       pl.BlockSpec((BLK, dout), lambda i: (i, 0))),
            out_shape=(jax.ShapeDtypeStruct((NC, NPAD, DH), jnp.float32)
                       if dout == D else
                       jax.ShapeDtypeStruct((NPAD, dout), jnp.float32)),
        )(a0, a1, g0, g1, dis, b, W)

    return run


_TC_MID128 = _make_tc_mid(D)
_TC_MID16 = _make_tc_mid(16)


def _tc_last(a0, a1, g2, dis, b2p):
    def body(a0_ref, a1_ref, g_ref, dis_ref, b_ref, o_ref):
        o_ref[...] = dis_ref[...] * (a0_ref[...] + a1_ref[...] + g_ref[...]) + b_ref[...]

    return pl.pallas_call(
        body,
        grid=(GRID,),
        in_specs=[
            pl.BlockSpec((BLK, 16), lambda i: (i, 0)),
            pl.BlockSpec((BLK, 16), lambda i: (i, 0)),
            pl.BlockSpec((BLK, 16), lambda i: (i, 0)),
            pl.BlockSpec((BLK, 1), lambda i: (i, 0)),
            pl.BlockSpec((1, 16), lambda i: (0, 0)),
        ],
        out_specs=pl.BlockSpec((BLK, 16), lambda i: (i, 0)),
        out_shape=jax.ShapeDtypeStruct((NPAD, 16), jnp.float32),
    )(a0, a1, g2, dis, b2p)


def kernel(x, edge_index, W0, b0, W1, b1, W2, b2):
    src = edge_index[0].astype(jnp.int32)
    dst = edge_index[1].astype(jnp.int32)
    pad_e = EPAD - E
    pad_ids = jnp.full((pad_e,), N, dtype=jnp.int32)
    src_p = jnp.concatenate([src, pad_ids])
    dst_p = jnp.concatenate([dst, pad_ids])
    dst3 = dst_p.reshape(NW, CPW, CHUNK)
    pk = src_p | (dst_p << 14)
    pk3 = pk.reshape(NW, CPW, CHUNK)
    pk16 = pk.reshape(NS, CPT, CHUNK)

    x_pad = jnp.pad(x, ((0, NPAD - N), (0, 0)))
    zeros64 = jnp.zeros((NPAD, DH), jnp.float32)
    zeros16 = jnp.zeros((NPAD, 16), jnp.float32)
    ones16 = jnp.ones((CHUNK, 16), jnp.float32)

    degp = _deg(dst3, ones16, zeros16)
    d0 = degp[0, :, 0:1]
    d1 = degp[1, :, 0:1]

    g0, dis = _tc_first(x_pad, d0, d1, W0)
    acc0 = _agg_col(g0, pk16, zeros64)
    g1 = _TC_MID128(acc0[0], acc0[1], g0[0], g0[1], dis, b0.reshape(1, D), W1)
    acc1 = _agg_col(g1, pk16, zeros64)

    W2p = jnp.pad(W2, ((0, 16 - W2.shape[0]), (0, 0)))
    b2p = jnp.pad(b2, (0, 16 - b2.shape[0]))
    g2 = _TC_MID16(acc1[0], acc1[1], g1[0], g1[1], dis, b1.reshape(1, D), W2p)
    acc2 = _AGG16(g2, pk3, zeros16)
    out16 = _tc_last(acc2[0], acc2[1], g2, dis, b2p.reshape(1, 16))
    return out16[:N, :5]


# speedup vs baseline: 1.1063x; 1.1063x over previous
import functools

import jax
import jax.numpy as jnp
from jax import lax
from jax.experimental import pallas as pl
from jax.experimental.pallas import tpu as pltpu
from jax.experimental.pallas import tpu_sc as plsc

N = 10000
E = 320000
D = 128
NPAD = 10240
NC, NS = 2, 16
NW = NC * NS
CHUNK = 128
CPW = 80
EPW = CPW * CHUNK
EPAD = NW * EPW
ROWS_T = NPAD // NS


def _mesh():
    return plsc.VectorSubcoreMesh(
        core_axis_name="c", subcore_axis_name="s", num_cores=NC, num_subcores=NS
    )


DH = D // NC
CPT = EPAD // CHUNK // NS


@functools.partial(
    pl.kernel,
    out_type=jax.ShapeDtypeStruct((NC, NPAD, DH), jnp.float32),
    mesh=_mesh(),
    compiler_params=pltpu.CompilerParams(use_tc_tiling_on_sc=False),
    scratch_types=[
        pltpu.VMEM((CPT, CHUNK), jnp.int32),
        pltpu.VMEM((CHUNK,), jnp.int32),
        pltpu.VMEM((CHUNK,), jnp.int32),
        pltpu.VMEM((CHUNK,), jnp.int32),
        pltpu.VMEM((CHUNK,), jnp.int32),
        pltpu.VMEM((CHUNK, DH), jnp.float32),
        pltpu.VMEM((CHUNK, DH), jnp.float32),
        pltpu.VMEM_SHARED((NPAD, DH), jnp.float32),
        pltpu.VMEM_SHARED((NPAD, DH), jnp.float32),
        pltpu.SemaphoreType.DMA,
        pltpu.SemaphoreType.DMA,
        pltpu.SemaphoreType.DMA,
        pltpu.SemaphoreType.DMA,
    ],
)
def _agg_col(g_hbm, pk_hbm, zero_hbm, out_hbm,
             pk_v, src_a, dst_a, src_b, dst_b, rows_a, rows_b,
             g_sh, acc_sh, gsa, gsb, ssa, ssb):
    c = lax.axis_index("c")
    s = lax.axis_index("s")
    row0 = s * ROWS_T
    d_zero = pltpu.async_copy(zero_hbm.at[pl.ds(row0, ROWS_T)],
                              acc_sh.at[pl.ds(row0, ROWS_T)], gsa)
    d_gst = pltpu.async_copy(g_hbm.at[c, pl.ds(row0, ROWS_T)],
                             g_sh.at[pl.ds(row0, ROWS_T)], gsb)
    d_pk = pltpu.async_copy(pk_hbm.at[s], pk_v, ssa)
    d_zero.wait()
    d_gst.wait()
    d_pk.wait()

    def unpack(ci, src_buf, dst_buf):
        for j in range(CHUNK // 16):
            p = pk_v[ci, pl.ds(16 * j, 16)]
            src_buf[pl.ds(16 * j, 16)] = p & (2**14 - 1)
            dst_buf[pl.ds(16 * j, 16)] = p >> 14

    plsc.subcore_barrier()

    unpack(0, src_a, dst_a)
    unpack(1, src_b, dst_b)
    pltpu.async_copy(g_sh.at[src_a], rows_a, gsa)
    pltpu.async_copy(g_sh.at[src_b], rows_b, gsb)

    def body(i, carry):
        a2 = jnp.minimum(2 * i + 2, CPT - 1)
        b2 = jnp.minimum(2 * i + 3, CPT - 1)
        pltpu.make_async_copy(g_sh.at[src_a], rows_a, gsa).wait()
        pltpu.async_copy(rows_a, acc_sh.at[dst_a], ssa, add=True)
        pltpu.make_async_copy(g_sh.at[src_b], rows_b, gsb).wait()
        pltpu.make_async_copy(rows_a, acc_sh.at[dst_a], ssa).wait()
        unpack(a2, src_a, dst_a)
        pltpu.async_copy(g_sh.at[src_a], rows_a, gsa)
        pltpu.async_copy(rows_b, acc_sh.at[dst_b], ssb, add=True)
        pltpu.make_async_copy(rows_b, acc_sh.at[dst_b], ssb).wait()
        unpack(b2, src_b, dst_b)
        pltpu.async_copy(g_sh.at[src_b], rows_b, gsb)
        return carry

    lax.fori_loop(0, CPT // 2, body, 0)
    pltpu.make_async_copy(g_sh.at[src_a], rows_a, gsa).wait()
    pltpu.make_async_copy(g_sh.at[src_b], rows_b, gsb).wait()
    plsc.subcore_barrier()
    pltpu.sync_copy(acc_sh.at[pl.ds(row0, ROWS_T)],
                    out_hbm.at[c, pl.ds(row0, ROWS_T)])


@functools.partial(
    pl.kernel,
    out_type=jax.ShapeDtypeStruct((NC, NPAD, 16), jnp.float32),
    mesh=_mesh(),
    compiler_params=pltpu.CompilerParams(use_tc_tiling_on_sc=False),
    scratch_types=[
        pltpu.VMEM((CPW, CHUNK), jnp.int32),
        pltpu.VMEM((CHUNK,), jnp.int32),
        pltpu.VMEM((CHUNK,), jnp.int32),
        pltpu.VMEM((CHUNK,), jnp.int32),
        pltpu.VMEM((CHUNK,), jnp.int32),
        pltpu.VMEM((CHUNK, 16), jnp.float32),
        pltpu.VMEM((CHUNK, 16), jnp.float32),
        pltpu.VMEM_SHARED((NPAD, 16), jnp.float32),
        pltpu.VMEM_SHARED((NPAD, 16), jnp.float32),
        pltpu.SemaphoreType.DMA,
        pltpu.SemaphoreType.DMA,
        pltpu.SemaphoreType.DMA,
        pltpu.SemaphoreType.DMA,
    ],
)
def _AGG16(g_hbm, pk_hbm, zero_hbm, out_hbm,
           pk_v, src_a, dst_a, src_b, dst_b, rows_a, rows_b,
           g_sh, acc_sh, gsa, gsb, ssa, ssb):
    c = lax.axis_index("c")
    s = lax.axis_index("s")
    w = s * NC + c
    row0 = s * ROWS_T
    d_zero = pltpu.async_copy(zero_hbm.at[pl.ds(row0, ROWS_T)],
                              acc_sh.at[pl.ds(row0, ROWS_T)], gsa)
    d_gst = pltpu.async_copy(g_hbm.at[pl.ds(row0, ROWS_T)],
                             g_sh.at[pl.ds(row0, ROWS_T)], gsb)
    d_pk = pltpu.async_copy(pk_hbm.at[w], pk_v, ssa)
    d_zero.wait()
    d_gst.wait()
    d_pk.wait()

    def unpack(ci, src_buf, dst_buf):
        for j in range(CHUNK // 16):
            p = pk_v[ci, pl.ds(16 * j, 16)]
            src_buf[pl.ds(16 * j, 16)] = p & (2**14 - 1)
            dst_buf[pl.ds(16 * j, 16)] = p >> 14

    plsc.subcore_barrier()

    unpack(0, src_a, dst_a)
    unpack(1, src_b, dst_b)
    pltpu.async_copy(g_sh.at[src_a], rows_a, gsa)
    pltpu.async_copy(g_sh.at[src_b], rows_b, gsb)

    def body(i, carry):
        a2 = jnp.minimum(2 * i + 2, CPW - 1)
        b2 = jnp.minimum(2 * i + 3, CPW - 1)
        pltpu.make_async_copy(g_sh.at[src_a], rows_a, gsa).wait()
        pltpu.async_copy(rows_a, acc_sh.at[dst_a], ssa, add=True)
        pltpu.make_async_copy(g_sh.at[src_b], rows_b, gsb).wait()
        pltpu.make_async_copy(rows_a, acc_sh.at[dst_a], ssa).wait()
        unpack(a2, src_a, dst_a)
        pltpu.async_copy(g_sh.at[src_a], rows_a, gsa)
        pltpu.async_copy(rows_b, acc_sh.at[dst_b], ssb, add=True)
        pltpu.make_async_copy(rows_b, acc_sh.at[dst_b], ssb).wait()
        unpack(b2, src_b, dst_b)
        pltpu.async_copy(g_sh.at[src_b], rows_b, gsb)
        return carry

    lax.fori_loop(0, CPW // 2, body, 0)
    pltpu.make_async_copy(g_sh.at[src_a], rows_a, gsa).wait()
    pltpu.make_async_copy(g_sh.at[src_b], rows_b, gsb).wait()
    plsc.subcore_barrier()
    pltpu.sync_copy(acc_sh.at[pl.ds(row0, ROWS_T)],
                    out_hbm.at[c, pl.ds(row0, ROWS_T)])


@functools.partial(
    pl.kernel,
    out_type=jax.ShapeDtypeStruct((NC, NPAD, 16), jnp.float32),
    mesh=_mesh(),
    compiler_params=pltpu.CompilerParams(use_tc_tiling_on_sc=False),
    scratch_types=[
        pltpu.VMEM((CPW, CHUNK), jnp.int32),
        pltpu.VMEM((CHUNK, 16), jnp.float32),
        pltpu.VMEM_SHARED((NPAD, 16), jnp.float32),
        pltpu.SemaphoreType.DMA,
    ],
)
def _deg(dst_hbm, ones_hbm, zero_hbm, out_hbm, dst_v, ones_v, acc_sh, sem):
    c = lax.axis_index("c")
    s = lax.axis_index("s")
    w = s * NC + c
    pltpu.sync_copy(zero_hbm.at[pl.ds(s * ROWS_T, ROWS_T)],
                    acc_sh.at[pl.ds(s * ROWS_T, ROWS_T)])
    pltpu.sync_copy(ones_hbm, ones_v)
    pltpu.sync_copy(dst_hbm.at[w], dst_v)
    plsc.subcore_barrier()

    def body(i, carry):
        for j in range(8):
            pltpu.async_copy(ones_v, acc_sh.at[dst_v.at[8 * i + j]], sem,
                             add=True)
        for j in range(8):
            pltpu.make_async_copy(ones_v, acc_sh.at[dst_v.at[8 * i + j]],
                                  sem).wait()
        return carry

    lax.fori_loop(0, CPW // 8, body, 0)
    plsc.subcore_barrier()
    pltpu.sync_copy(acc_sh.at[pl.ds(s * ROWS_T, ROWS_T)],
                    out_hbm.at[c, pl.ds(s * ROWS_T, ROWS_T)])


BLK = 512
GRID = NPAD // BLK


def _tc_first(x_pad, d0, d1, W0):
    def body(x_ref, d0_ref, d1_ref, w_ref, g_ref, dis_ref):
        dis = lax.rsqrt(1.0 + d0_ref[...] + d1_ref[...])
        h = lax.dot_general(x_ref[...], w_ref[...], (((1,), (1,)), ((), ())),
                            preferred_element_type=jnp.float32)
        hd = h * dis
        g_ref[...] = jnp.stack([hd[:, :DH], hd[:, DH:]])
        dis_ref[...] = dis

    return pl.pallas_call(
        body,
        grid=(GRID,),
        in_specs=[
            pl.BlockSpec((BLK, D), lambda i: (i, 0)),
            pl.BlockSpec((BLK, 1), lambda i: (i, 0)),
            pl.BlockSpec((BLK, 1), lambda i: (i, 0)),
            pl.BlockSpec((D, D), lambda i: (0, 0)),
        ],
        out_specs=[
            pl.BlockSpec((NC, BLK, DH), lambda i: (0, i, 0)),
            pl.BlockSpec((BLK, 1), lambda i: (i, 0)),
        ],
        out_shape=[
            jax.ShapeDtypeStruct((NC, NPAD, DH), jnp.float32),
            jax.ShapeDtypeStruct((NPAD, 1), jnp.float32),
        ],
    )(x_pad, d0, d1, W0)


def _make_tc_mid(dout):
    def body(a0_ref, a1_ref, g0_ref, g1_ref, dis_ref, b_ref, w_ref, o_ref):
        dis = dis_ref[...]
        t = jnp.concatenate([a0_ref[...] + g0_ref[...],
                             a1_ref[...] + g1_ref[...]], axis=1)
        x = jnp.maximum(dis * t + b_ref[...], 0.0)
        h = lax.dot_general(x, w_ref[...], (((1,), (1,)), ((), ())),
                            preferred_element_type=jnp.float32)
        hd = h * dis
        if dout == D:
            o_ref[...] = jnp.stack([hd[:, :DH], hd[:, DH:]])
        else:
            o_ref[...] = hd

    def run(a0, a1, g0, g1, dis, b, W):
        return pl.pallas_call(
            body,
            grid=(GRID,),
            in_specs=[
                pl.BlockSpec((BLK, DH), lambda i: (i, 0)),
                pl.BlockSpec((BLK, DH), lambda i: (i, 0)),
                pl.BlockSpec((BLK, DH), lambda i: (i, 0)),
                pl.BlockSpec((BLK, DH), lambda i: (i, 0)),
                pl.BlockSpec((BLK, 1), lambda i: (i, 0)),
                pl.BlockSpec((1, D), lambda i: (0, 0)),
                pl.BlockSpec((dout, D), lambda i: (0, 0)),
            ],
            out_specs=(pl.BlockSpec((NC, BLK, DH), lambda i: (0, i, 0))
                       if dout == D else
                       pl.BlockSpec((BLK, dout), lambda i: (i, 0))),
            out_shape=(jax.ShapeDtypeStruct((NC, NPAD, DH), jnp.float32)
                       if dout == D else
                       jax.ShapeDtypeStruct((NPAD, dout), jnp.float32)),
        )(a0, a1, g0, g1, dis, b, W)

    return run


_TC_MID128 = _make_tc_mid(D)
_TC_MID16 = _make_tc_mid(16)


def _tc_last(a0, a1, g2, dis, b2p):
    def body(a0_ref, a1_ref, g_ref, dis_ref, b_ref, o_ref):
        o_ref[...] = dis_ref[...] * (a0_ref[...] + a1_ref[...] + g_ref[...]) + b_ref[...]

    return pl.pallas_call(
        body,
        grid=(GRID,),
        in_specs=[
            pl.BlockSpec((BLK, 16), lambda i: (i, 0)),
            pl.BlockSpec((BLK, 16), lambda i: (i, 0)),
            pl.BlockSpec((BLK, 16), lambda i: (i, 0)),
            pl.BlockSpec((BLK, 1), lambda i: (i, 0)),
            pl.BlockSpec((1, 16), lambda i: (0, 0)),
        ],
        out_specs=pl.BlockSpec((BLK, 16), lambda i: (i, 0)),
        out_shape=jax.ShapeDtypeStruct((NPAD, 16), jnp.float32),
    )(a0, a1, g2, dis, b2p)


def kernel(x, edge_index, W0, b0, W1, b1, W2, b2):
    src = edge_index[0].astype(jnp.int32)
    dst = edge_index[1].astype(jnp.int32)
    pad_e = EPAD - E
    pad_ids = jnp.full((pad_e,), N, dtype=jnp.int32)
    src_p = jnp.concatenate([src, pad_ids])
    dst_p = jnp.concatenate([dst, pad_ids])
    dst3 = dst_p.reshape(NW, CPW, CHUNK)
    pk = src_p | (dst_p << 14)
    pk3 = pk.reshape(NW, CPW, CHUNK)
    pk16 = pk.reshape(NS, CPT, CHUNK)

    x_pad = jnp.pad(x, ((0, NPAD - N), (0, 0)))
    zeros64 = jnp.zeros((NPAD, DH), jnp.float32)
    zeros16 = jnp.zeros((NPAD, 16), jnp.float32)
    ones16 = jnp.ones((CHUNK, 16), jnp.float32)

    degp = _deg(dst3, ones16, zeros16)
    d0 = degp[0, :, 0:1]
    d1 = degp[1, :, 0:1]

    g0, dis = _tc_first(x_pad, d0, d1, W0)
    acc0 = _agg_col(g0, pk16, zeros64)
    g1 = _TC_MID128(acc0[0], acc0[1], g0[0], g0[1], dis, b0.reshape(1, D), W1)
    acc1 = _agg_col(g1, pk16, zeros64)

    W2p = jnp.pad(W2, ((0, 16 - W2.shape[0]), (0, 0)))
    b2p = jnp.pad(b2, (0, 16 - b2.shape[0]))
    g2 = _TC_MID16(acc1[0], acc1[1], g1[0], g1[1], dis, b1.reshape(1, D), W2p)
    acc2 = _AGG16(g2, pk3, zeros16)
    out16 = _tc_last(acc2[0], acc2[1], g2, dis, b2p.reshape(1, 16))
    return out16[:N, :5]


# speedup vs baseline: 1.1080x; 1.0016x over previous
import functools

import jax
import jax.numpy as jnp
from jax import lax
from jax.experimental import pallas as pl
from jax.experimental.pallas import tpu as pltpu
from jax.experimental.pallas import tpu_sc as plsc

N = 10000
E = 320000
D = 128
NPAD = 10240
NC, NS = 2, 16
NW = NC * NS
CHUNK = 128
CPW = 80
EPW = CPW * CHUNK
EPAD = NW * EPW
ROWS_T = NPAD // NS


def _mesh():
    return plsc.VectorSubcoreMesh(
        core_axis_name="c", subcore_axis_name="s", num_cores=NC, num_subcores=NS
    )


DH = D // NC
CPT = EPAD // CHUNK // NS


@functools.partial(
    pl.kernel,
    out_type=jax.ShapeDtypeStruct((NC, NPAD, DH), jnp.float32),
    mesh=_mesh(),
    compiler_params=pltpu.CompilerParams(use_tc_tiling_on_sc=False),
    scratch_types=[
        pltpu.VMEM((CPT, CHUNK), jnp.int32),
        pltpu.VMEM((CHUNK,), jnp.int32),
        pltpu.VMEM((CHUNK,), jnp.int32),
        pltpu.VMEM((CHUNK,), jnp.int32),
        pltpu.VMEM((CHUNK,), jnp.int32),
        pltpu.VMEM((CHUNK, DH), jnp.float32),
        pltpu.VMEM((CHUNK, DH), jnp.float32),
        pltpu.VMEM_SHARED((NPAD, DH), jnp.float32),
        pltpu.VMEM_SHARED((NPAD, DH), jnp.float32),
        pltpu.SemaphoreType.DMA,
        pltpu.SemaphoreType.DMA,
        pltpu.SemaphoreType.DMA,
        pltpu.SemaphoreType.DMA,
    ],
)
def _agg_col(g_hbm, pk_hbm, zero_hbm, out_hbm,
             pk_v, src_a, dst_a, src_b, dst_b, rows_a, rows_b,
             g_sh, acc_sh, gsa, gsb, ssa, ssb):
    c = lax.axis_index("c")
    s = lax.axis_index("s")
    row0 = s * ROWS_T
    d_zero = pltpu.async_copy(zero_hbm.at[pl.ds(row0, ROWS_T)],
                              acc_sh.at[pl.ds(row0, ROWS_T)], gsa)
    d_gst = pltpu.async_copy(g_hbm.at[c, pl.ds(row0, ROWS_T)],
                             g_sh.at[pl.ds(row0, ROWS_T)], gsb)
    d_pk = pltpu.async_copy(pk_hbm.at[s], pk_v, ssa)
    d_zero.wait()
    d_gst.wait()
    d_pk.wait()

    def unpack(ci, src_buf, dst_buf):
        for j in range(CHUNK // 16):
            p = pk_v[ci, pl.ds(16 * j, 16)]
            src_buf[pl.ds(16 * j, 16)] = p & (2**14 - 1)
            dst_buf[pl.ds(16 * j, 16)] = p >> 14

    plsc.subcore_barrier()

    unpack(0, src_a, dst_a)
    unpack(1, src_b, dst_b)
    pltpu.async_copy(g_sh.at[src_a], rows_a, gsa)
    pltpu.async_copy(g_sh.at[src_b], rows_b, gsb)

    def body(i, carry):
        a2 = jnp.minimum(2 * i + 2, CPT - 1)
        b2 = jnp.minimum(2 * i + 3, CPT - 1)
        pltpu.make_async_copy(g_sh.at[src_a], rows_a, gsa).wait()
        pltpu.async_copy(rows_a, acc_sh.at[dst_a], ssa, add=True)
        pltpu.make_async_copy(g_sh.at[src_b], rows_b, gsb).wait()
        pltpu.make_async_copy(rows_a, acc_sh.at[dst_a], ssa).wait()
        unpack(a2, src_a, dst_a)
        pltpu.async_copy(g_sh.at[src_a], rows_a, gsa)
        pltpu.async_copy(rows_b, acc_sh.at[dst_b], ssb, add=True)
        pltpu.make_async_copy(rows_b, acc_sh.at[dst_b], ssb).wait()
        unpack(b2, src_b, dst_b)
        pltpu.async_copy(g_sh.at[src_b], rows_b, gsb)
        return carry

    lax.fori_loop(0, CPT // 2, body, 0)
    pltpu.make_async_copy(g_sh.at[src_a], rows_a, gsa).wait()
    pltpu.make_async_copy(g_sh.at[src_b], rows_b, gsb).wait()
    plsc.subcore_barrier()
    pltpu.sync_copy(acc_sh.at[pl.ds(row0, ROWS_T)],
                    out_hbm.at[c, pl.ds(row0, ROWS_T)])


@functools.partial(
    pl.kernel,
    out_type=jax.ShapeDtypeStruct((NC, NPAD, 16), jnp.float32),
    mesh=_mesh(),
    compiler_params=pltpu.CompilerParams(use_tc_tiling_on_sc=False),
    scratch_types=[
        pltpu.VMEM((CPW, CHUNK), jnp.int32),
        pltpu.VMEM((CHUNK,), jnp.int32),
        pltpu.VMEM((CHUNK,), jnp.int32),
        pltpu.VMEM((CHUNK,), jnp.int32),
        pltpu.VMEM((CHUNK,), jnp.int32),
        pltpu.VMEM((CHUNK, 16), jnp.float32),
        pltpu.VMEM((CHUNK, 16), jnp.float32),
        pltpu.VMEM_SHARED((NPAD, 16), jnp.float32),
        pltpu.VMEM_SHARED((NPAD, 16), jnp.float32),
        pltpu.SemaphoreType.DMA,
        pltpu.SemaphoreType.DMA,
        pltpu.SemaphoreType.DMA,
        pltpu.SemaphoreType.DMA,
    ],
)
def _AGG16(g_hbm, pk_hbm, zero_hbm, out_hbm,
           pk_v, src_a, dst_a, src_b, dst_b, rows_a, rows_b,
           g_sh, acc_sh, gsa, gsb, ssa, ssb):
    c = lax.axis_index("c")
    s = lax.axis_index("s")
    w = s * NC + c
    row0 = s * ROWS_T
    d_zero = pltpu.async_copy(zero_hbm.at[pl.ds(row0, ROWS_T)],
                              acc_sh.at[pl.ds(row0, ROWS_T)], gsa)
    d_gst = pltpu.async_copy(g_hbm.at[pl.ds(row0, ROWS_T)],
                             g_sh.at[pl.ds(row0, ROWS_T)], gsb)
    d_pk = pltpu.async_copy(pk_hbm.at[w], pk_v, ssa)
    d_zero.wait()
    d_gst.wait()
    d_pk.wait()

    def unpack(ci, src_buf, dst_buf):
        for j in range(CHUNK // 16):
            p = pk_v[ci, pl.ds(16 * j, 16)]
            src_buf[pl.ds(16 * j, 16)] = p & (2**14 - 1)
            dst_buf[pl.ds(16 * j, 16)] = p >> 14

    plsc.subcore_barrier()

    unpack(0, src_a, dst_a)
    unpack(1, src_b, dst_b)
    pltpu.async_copy(g_sh.at[src_a], rows_a, gsa)
    pltpu.async_copy(g_sh.at[src_b], rows_b, gsb)

    def body(i, carry):
        a2 = jnp.minimum(2 * i + 2, CPW - 1)
        b2 = jnp.minimum(2 * i + 3, CPW - 1)
        pltpu.make_async_copy(g_sh.at[src_a], rows_a, gsa).wait()
        pltpu.async_copy(rows_a, acc_sh.at[dst_a], ssa, add=True)
        pltpu.make_async_copy(g_sh.at[src_b], rows_b, gsb).wait()
        pltpu.make_async_copy(rows_a, acc_sh.at[dst_a], ssa).wait()
        unpack(a2, src_a, dst_a)
        pltpu.async_copy(g_sh.at[src_a], rows_a, gsa)
        pltpu.async_copy(rows_b, acc_sh.at[dst_b], ssb, add=True)
        pltpu.make_async_copy(rows_b, acc_sh.at[dst_b], ssb).wait()
        unpack(b2, src_b, dst_b)
        pltpu.async_copy(g_sh.at[src_b], rows_b, gsb)
        return carry

    lax.fori_loop(0, CPW // 2, body, 0)
    pltpu.make_async_copy(g_sh.at[src_a], rows_a, gsa).wait()
    pltpu.make_async_copy(g_sh.at[src_b], rows_b, gsb).wait()
    plsc.subcore_barrier()
    pltpu.sync_copy(acc_sh.at[pl.ds(row0, ROWS_T)],
                    out_hbm.at[c, pl.ds(row0, ROWS_T)])


@functools.partial(
    pl.kernel,
    out_type=jax.ShapeDtypeStruct((NC, NPAD, 16), jnp.float32),
    mesh=_mesh(),
    compiler_params=pltpu.CompilerParams(use_tc_tiling_on_sc=False),
    scratch_types=[
        pltpu.VMEM((CPW, CHUNK), jnp.int32),
        pltpu.VMEM((CHUNK, 16), jnp.float32),
        pltpu.VMEM_SHARED((NPAD, 16), jnp.float32),
        pltpu.SemaphoreType.DMA,
    ],
)
def _deg(dst_hbm, ones_hbm, zero_hbm, out_hbm, dst_v, ones_v, acc_sh, sem):
    c = lax.axis_index("c")
    s = lax.axis_index("s")
    w = s * NC + c
    pltpu.sync_copy(zero_hbm.at[pl.ds(s * ROWS_T, ROWS_T)],
                    acc_sh.at[pl.ds(s * ROWS_T, ROWS_T)])
    pltpu.sync_copy(ones_hbm, ones_v)
    pltpu.sync_copy(dst_hbm.at[w], dst_v)
    plsc.subcore_barrier()

    def body(i, carry):
        for j in range(8):
            pltpu.async_copy(ones_v, acc_sh.at[dst_v.at[8 * i + j]], sem,
                             add=True)
        for j in range(8):
            pltpu.make_async_copy(ones_v, acc_sh.at[dst_v.at[8 * i + j]],
                                  sem).wait()
        return carry

    lax.fori_loop(0, CPW // 8, body, 0)
    plsc.subcore_barrier()
    pltpu.sync_copy(acc_sh.at[pl.ds(s * ROWS_T, ROWS_T)],
                    out_hbm.at[c, pl.ds(s * ROWS_T, ROWS_T)])


BLK = 512
GRID = NPAD // BLK


def _tc_h0(x_pad, W0):
    def body(x_ref, w_ref, h_ref):
        h_ref[...] = lax.dot_general(x_ref[...], w_ref[...],
                                     (((1,), (1,)), ((), ())),
                                     preferred_element_type=jnp.float32)

    return pl.pallas_call(
        body,
        grid=(GRID,),
        in_specs=[
            pl.BlockSpec((BLK, D), lambda i: (i, 0)),
            pl.BlockSpec((D, D), lambda i: (0, 0)),
        ],
        out_specs=pl.BlockSpec((BLK, D), lambda i: (i, 0)),
        out_shape=jax.ShapeDtypeStruct((NPAD, D), jnp.float32),
    )(x_pad, W0)


def _tc_first(h0, d0, d1):
    def body(h_ref, d0_ref, d1_ref, g_ref, dis_ref):
        dis = lax.rsqrt(1.0 + d0_ref[...] + d1_ref[...])
        hd = h_ref[...] * dis
        g_ref[...] = jnp.stack([hd[:, :DH], hd[:, DH:]])
        dis_ref[...] = dis

    return pl.pallas_call(
        body,
        grid=(GRID,),
        in_specs=[
            pl.BlockSpec((BLK, D), lambda i: (i, 0)),
            pl.BlockSpec((BLK, 1), lambda i: (i, 0)),
            pl.BlockSpec((BLK, 1), lambda i: (i, 0)),
        ],
        out_specs=[
            pl.BlockSpec((NC, BLK, DH), lambda i: (0, i, 0)),
            pl.BlockSpec((BLK, 1), lambda i: (i, 0)),
        ],
        out_shape=[
            jax.ShapeDtypeStruct((NC, NPAD, DH), jnp.float32),
            jax.ShapeDtypeStruct((NPAD, 1), jnp.float32),
        ],
    )(h0, d0, d1)


def _make_tc_mid(dout):
    def body(a0_ref, a1_ref, g0_ref, g1_ref, dis_ref, b_ref, w_ref, o_ref):
        dis = dis_ref[...]
        t = jnp.concatenate([a0_ref[...] + g0_ref[...],
                             a1_ref[...] + g1_ref[...]], axis=1)
        x = jnp.maximum(dis * t + b_ref[...], 0.0)
        h = lax.dot_general(x, w_ref[...], (((1,), (1,)), ((), ())),
                            preferred_element_type=jnp.float32)
        hd = h * dis
        if dout == D:
            o_ref[...] = jnp.stack([hd[:, :DH], hd[:, DH:]])
        else:
            o_ref[...] = hd

    def run(a0, a1, g0, g1, dis, b, W):
        return pl.pallas_call(
            body,
            grid=(GRID,),
            in_specs=[
                pl.BlockSpec((BLK, DH), lambda i: (i, 0)),
                pl.BlockSpec((BLK, DH), lambda i: (i, 0)),
                pl.BlockSpec((BLK, DH), lambda i: (i, 0)),
                pl.BlockSpec((BLK, DH), lambda i: (i, 0)),
                pl.BlockSpec((BLK, 1), lambda i: (i, 0)),
                pl.BlockSpec((1, D), lambda i: (0, 0)),
                pl.BlockSpec((dout, D), lambda i: (0, 0)),
            ],
            out_specs=(pl.BlockSpec((NC, BLK, DH), lambda i: (0, i, 0))
                       if dout == D else
                       pl.BlockSpec((BLK, dout), lambda i: (i, 0))),
            out_shape=(jax.ShapeDtypeStruct((NC, NPAD, DH), jnp.float32)
                       if dout == D else
                       jax.ShapeDtypeStruct((NPAD, dout), jnp.float32)),
        )(a0, a1, g0, g1, dis, b, W)

    return run


_TC_MID128 = _make_tc_mid(D)
_TC_MID16 = _make_tc_mid(16)


def _tc_last(a0, a1, g2, dis, b2p):
    def body(a0_ref, a1_ref, g_ref, dis_ref, b_ref, o_ref):
        o_ref[...] = dis_ref[...] * (a0_ref[...] + a1_ref[...] + g_ref[...]) + b_ref[...]

    return pl.pallas_call(
        body,
        grid=(GRID,),
        in_specs=[
            pl.BlockSpec((BLK, 16), lambda i: (i, 0)),
            pl.BlockSpec((BLK, 16), lambda i: (i, 0)),
            pl.BlockSpec((BLK, 16), lambda i: (i, 0)),
            pl.BlockSpec((BLK, 1), lambda i: (i, 0)),
            pl.BlockSpec((1, 16), lambda i: (0, 0)),
        ],
        out_specs=pl.BlockSpec((BLK, 16), lambda i: (i, 0)),
        out_shape=jax.ShapeDtypeStruct((NPAD, 16), jnp.float32),
    )(a0, a1, g2, dis, b2p)


def kernel(x, edge_index, W0, b0, W1, b1, W2, b2):
    src = edge_index[0].astype(jnp.int32)
    dst = edge_index[1].astype(jnp.int32)
    pad_e = EPAD - E
    pad_ids = jnp.full((pad_e,), N, dtype=jnp.int32)
    src_p = jnp.concatenate([src, pad_ids])
    dst_p = jnp.concatenate([dst, pad_ids])
    dst3 = dst_p.reshape(NW, CPW, CHUNK)
    pk = src_p | (dst_p << 14)
    pk3 = pk.reshape(NW, CPW, CHUNK)
    pk16 = pk.reshape(NS, CPT, CHUNK)

    x_pad = jnp.pad(x, ((0, NPAD - N), (0, 0)))
    zeros64 = jnp.zeros((NPAD, DH), jnp.float32)
    zeros16 = jnp.zeros((NPAD, 16), jnp.float32)
    ones16 = jnp.ones((CHUNK, 16), jnp.float32)

    h0 = _tc_h0(x_pad, W0)
    degp = _deg(dst3, ones16, zeros16)
    d0 = degp[0, :, 0:1]
    d1 = degp[1, :, 0:1]

    g0, dis = _tc_first(h0, d0, d1)
    acc0 = _agg_col(g0, pk16, zeros64)
    g1 = _TC_MID128(acc0[0], acc0[1], g0[0], g0[1], dis, b0.reshape(1, D), W1)
    acc1 = _agg_col(g1, pk16, zeros64)

    W2p = jnp.pad(W2, ((0, 16 - W2.shape[0]), (0, 0)))
    b2p = jnp.pad(b2, (0, 16 - b2.shape[0]))
    g2 = _TC_MID16(acc1[0], acc1[1], g1[0], g1[1], dis, b1.reshape(1, D), W2p)
    acc2 = _AGG16(g2, pk3, zeros16)
    out16 = _tc_last(acc2[0], acc2[1], g2, dis, b2p.reshape(1, 16))
    return out16[:N, :5]
